# baseline (speedup 1.0000x reference)
import jax
import jax.numpy as jnp
from jax import lax
from jax.experimental import pallas as pl
from jax.experimental.pallas import tpu as pltpu
from jax.experimental.pallas import tpu_sc as plsc

N, C, H, W = 8, 128, 224, 224
PH, PW = H + 2, W + 2
P = H * W
TOTAL = N * P
CW = C // 2
PPR = PW // 2
PAIRS_PER_IMG = PH * PPR
NPAIR = N * PAIRS_PER_IMG
NC, NS = 2, 16
NW = NC * NS
PER_TILE = TOTAL // NW
CHUNK = 128
NCHUNK = PER_TILE // CHUNK
PAIRS = NCHUNK // 2
SUBS = CHUNK // 16
L = 16
OBW = 2 * CHUNK * CW


def _sc_body(table, xs_hbm, ys_hbm, out_hbm,
             xs_v, ys_v,
             iy00, iy10, w00, w10, w20, w30, b00, b10,
             iy01, iy11, w01, w11, w21, w31, b01, b11,
             ob,
             s00, s10, s01, s11, so):
    setA = (iy00, iy10, w00, w10, w20, w30, b00, b10, s00, s10)
    setB = (iy01, iy11, w01, w11, w21, w31, b01, b11, s01, s11)

    wid = lax.axis_index("s") * NC + lax.axis_index("c")
    base_g = wid * PER_TILE
    row_base = (wid // 4) * PAIRS_PER_IMG

    pltpu.sync_copy(xs_hbm.at[pl.ds(base_g, PER_TILE)], xs_v)
    pltpu.sync_copy(ys_hbm.at[pl.ds(base_g, PER_TILE)], ys_v)

    def fire(chk, S):
        i0, i1, wa_v, wb_v, wc_v, wd_v, b0, b1, s0, s1 = S
        off = chk * CHUNK
        for s in range(SUBS):
            xv = xs_v[pl.ds(off + s * L, L)]
            yv = ys_v[pl.ds(off + s * L, L)]
            x = ((xv + 1.0) * W - 1.0) * 0.5
            y = ((yv + 1.0) * H - 1.0) * 0.5
            xi = x.astype(jnp.int32)
            yi = y.astype(jnp.int32)
            x0 = jnp.where(xi.astype(jnp.float32) > x, xi - 1, xi)
            y0 = jnp.where(yi.astype(jnp.float32) > y, yi - 1, yi)
            x0f = x0.astype(jnp.float32)
            y0f = y0.astype(jnp.float32)
            dx1 = (x0f + 1.0) - x
            dx0 = x - x0f
            dy1 = (y0f + 1.0) - y
            dy0 = y - y0f
            xp = x0 + 1
            flat = (y0 + 1) * PW + xp
            q = row_base + lax.shift_right_logical(flat, 1) \
                + (xp & 1) * NPAIR
            i0[pl.ds(s * L, L)] = q
            i1[pl.ds(s * L, L)] = q + PPR
            wa_v[pl.ds(s * L, L)] = dx1 * dy1
            wb_v[pl.ds(s * L, L)] = dx1 * dy0
            wc_v[pl.ds(s * L, L)] = dx0 * dy1
            wd_v[pl.ds(s * L, L)] = dx0 * dy0
        pltpu.async_copy(table.at[i0], b0, s0)
        pltpu.async_copy(table.at[i1], b1, s1)

    def wait_gathers(S):
        i0, i1, _, _, _, _, b0, b1, s0, s1 = S
        pltpu.make_async_copy(table.at[i0], b0, s0).wait()
        pltpu.make_async_copy(table.at[i1], b1, s1).wait()

    def combine(S, half):
        _, _, wa_v, wb_v, wc_v, wd_v, b0, b1, _, _ = S
        fmt = plsc.PackFormat.INTERLEAVED

        def p_body(p, c2):
            pv = jnp.full((L,), 0, jnp.int32) + p
            wav = plsc.load_gather(wa_v, [pv])
            wbv = plsc.load_gather(wb_v, [pv])
            wcv = plsc.load_gather(wc_v, [pv])
            wdv = plsc.load_gather(wd_v, [pv])
            wa2 = plsc.pack(wav, wav, format=fmt)
            wb2 = plsc.pack(wbv, wbv, format=fmt)
            wc2 = plsc.pack(wcv, wcv, format=fmt)
            wd2 = plsc.pack(wdv, wdv, format=fmt)
            qw = (p + half * CHUNK) * CW
            for w4 in range(CW // L):
                sl0 = pl.ds(w4 * L, L)
                sl1 = pl.ds(CW + w4 * L, L)
                va = plsc.bitcast(b0[p, sl0], jnp.bfloat16)
                vc = plsc.bitcast(b0[p, sl1], jnp.bfloat16)
                vb = plsc.bitcast(b1[p, sl0], jnp.bfloat16)
                vd = plsc.bitcast(b1[p, sl1], jnp.bfloat16)
                acc = (va * wa2 + vc * wc2) + (vb * wb2 + vd * wd2)
                ob[pl.ds(qw + w4 * L, L)] = plsc.bitcast(acc, jnp.int32)
            return c2

        lax.fori_loop(0, CHUNK, p_body, 0, unroll=2)

    def out_copy(k):
        return pltpu.make_async_copy(
            ob, out_hbm.at[pl.ds(base_g * CW + k * OBW, OBW)], so)

    fire(0, setA)
    out_copy(0).start()

    def step(k, carry):
        c0 = 2 * k
        fire(c0 + 1, setB)
        wait_gathers(setA)
        out_copy(k).wait()
        combine(setA, 0)
        nxt = jnp.minimum(c0 + 2, NCHUNK - 2)
        fire(nxt, setA)
        wait_gathers(setB)
        combine(setB, 1)
        out_copy(k).start()
        return carry

    lax.fori_loop(0, PAIRS, step, 0)

    out_copy(0).wait()
    wait_gathers(setA)


def _scratch_set():
    return (
        [pltpu.VMEM((CHUNK,), jnp.int32) for _ in range(2)]
        + [pltpu.VMEM((CHUNK,), jnp.float32) for _ in range(4)]
        + [pltpu.VMEM((CHUNK, 2 * CW), jnp.int32) for _ in range(2)]
    )


_sc_sample = pl.kernel(
    _sc_body,
    out_type=jax.ShapeDtypeStruct((TOTAL * CW,), jnp.int32),
    mesh=plsc.VectorSubcoreMesh(
        core_axis_name="c", subcore_axis_name="s",
        num_cores=NC, num_subcores=NS),
    compiler_params=pltpu.CompilerParams(needs_layout_passes=False),
    scratch_types=(
        [pltpu.VMEM((PER_TILE,), jnp.float32),
         pltpu.VMEM((PER_TILE,), jnp.float32)]
        + _scratch_set()
        + _scratch_set()
        + [pltpu.VMEM((OBW,), jnp.int32)]
        + [pltpu.SemaphoreType.DMA] * 5
    ),
)


@jax.jit
def kernel(img, points):
    n, c, h, w = img.shape
    tl = jnp.pad(
        img.transpose(0, 2, 3, 1), ((0, 0), (1, 1), (1, 1), (0, 0))
    ).astype(jnp.bfloat16)
    fw = jax.lax.bitcast_convert_type(tl.reshape(-1, 2), jnp.int32)
    even = fw.reshape(NPAIR, 2 * CW)
    odd = jnp.concatenate(
        [fw[CW:], jnp.zeros((CW,), jnp.int32)]).reshape(NPAIR, 2 * CW)
    table = jnp.concatenate([even, odd], axis=0)
    xs = points[..., 0].reshape(-1)
    ys = points[..., 1].reshape(-1)
    out_w = _sc_sample(table, xs, ys)
    out_t = jax.lax.bitcast_convert_type(
        out_w, jnp.bfloat16).reshape(n, h, w, c)
    return out_t.transpose(0, 3, 1, 2).astype(jnp.float32)


# speedup vs baseline: 13.9863x; 13.9863x over previous
import jax
import jax.numpy as jnp
from jax import lax
from jax.experimental import pallas as pl
from jax.experimental.pallas import tpu as pltpu
from jax.experimental.pallas import tpu_sc as plsc

N, C, H, W = 8, 128, 224, 224
PH, PW = H + 2, W + 2
P = H * W
TOTAL = N * P
CW = C // 2
PPR = PW // 2
PAIRS_PER_IMG = PH * PPR
NPAIR = N * PAIRS_PER_IMG
NC, NS = 2, 16
NW = NC * NS
PER_TILE = TOTAL // NW
CHUNK = 128
NCHUNK = PER_TILE // CHUNK
PAIRS = NCHUNK // 2
SUBS = CHUNK // 16
L = 16
OBW = 2 * CHUNK * CW


def _sc_body(table, xs_hbm, ys_hbm, out_hbm,
             xs_v, ys_v,
             iy00, iy10, w00, w10, w20, w30, b00, b10,
             iy01, iy11, w01, w11, w21, w31, b01, b11,
             ob,
             s00, s10, s01, s11, so):
    setA = (iy00, iy10, w00, w10, w20, w30, b00, b10, s00, s10)
    setB = (iy01, iy11, w01, w11, w21, w31, b01, b11, s01, s11)

    wid = lax.axis_index("s") * NC + lax.axis_index("c")
    base_g = wid * PER_TILE
    row_base = (wid // 4) * PAIRS_PER_IMG

    pltpu.sync_copy(xs_hbm.at[pl.ds(base_g, PER_TILE)], xs_v)
    pltpu.sync_copy(ys_hbm.at[pl.ds(base_g, PER_TILE)], ys_v)

    def fire(chk, S):
        i0, i1, wa_v, wb_v, wc_v, wd_v, b0, b1, s0, s1 = S
        off = chk * CHUNK
        for s in range(SUBS):
            xv = xs_v[pl.ds(off + s * L, L)]
            yv = ys_v[pl.ds(off + s * L, L)]
            x = ((xv + 1.0) * W - 1.0) * 0.5
            y = ((yv + 1.0) * H - 1.0) * 0.5
            xi = x.astype(jnp.int32)
            yi = y.astype(jnp.int32)
            x0 = jnp.where(xi.astype(jnp.float32) > x, xi - 1, xi)
            y0 = jnp.where(yi.astype(jnp.float32) > y, yi - 1, yi)
            x0f = x0.astype(jnp.float32)
            y0f = y0.astype(jnp.float32)
            dx1 = (x0f + 1.0) - x
            dx0 = x - x0f
            dy1 = (y0f + 1.0) - y
            dy0 = y - y0f
            xp = x0 + 1
            flat = (y0 + 1) * PW + xp
            q = row_base + lax.shift_right_logical(flat, 1) \
                + (xp & 1) * NPAIR
            i0[pl.ds(s * L, L)] = q
            i1[pl.ds(s * L, L)] = q + PPR
            wa_v[pl.ds(s * L, L)] = dx1 * dy1
            wb_v[pl.ds(s * L, L)] = dx1 * dy0
            wc_v[pl.ds(s * L, L)] = dx0 * dy1
            wd_v[pl.ds(s * L, L)] = dx0 * dy0
        pltpu.async_copy(table.at[i0], b0, s0)
        pltpu.async_copy(table.at[i1], b1, s1)

    def wait_gathers(S):
        i0, i1, _, _, _, _, b0, b1, s0, s1 = S
        pltpu.make_async_copy(table.at[i0], b0, s0).wait()
        pltpu.make_async_copy(table.at[i1], b1, s1).wait()

    def combine(S, half):
        _, _, wa_v, wb_v, wc_v, wd_v, b0, b1, _, _ = S
        fmt = plsc.PackFormat.INTERLEAVED

        def p_body(p, c2):
            pv = jnp.full((L,), 0, jnp.int32) + p
            wav = plsc.load_gather(wa_v, [pv])
            wbv = plsc.load_gather(wb_v, [pv])
            wcv = plsc.load_gather(wc_v, [pv])
            wdv = plsc.load_gather(wd_v, [pv])
            wa2 = plsc.pack(wav, wav, format=fmt)
            wb2 = plsc.pack(wbv, wbv, format=fmt)
            wc2 = plsc.pack(wcv, wcv, format=fmt)
            wd2 = plsc.pack(wdv, wdv, format=fmt)
            qw = (p + half * CHUNK) * CW
            for w4 in range(CW // L):
                sl0 = pl.ds(w4 * L, L)
                sl1 = pl.ds(CW + w4 * L, L)
                va = plsc.bitcast(b0[p, sl0], jnp.bfloat16)
                vc = plsc.bitcast(b0[p, sl1], jnp.bfloat16)
                vb = plsc.bitcast(b1[p, sl0], jnp.bfloat16)
                vd = plsc.bitcast(b1[p, sl1], jnp.bfloat16)
                acc = (va * wa2 + vc * wc2) + (vb * wb2 + vd * wd2)
                ob[pl.ds(qw + w4 * L, L)] = plsc.bitcast(acc, jnp.int32)
            return c2

        lax.fori_loop(0, CHUNK, p_body, 0, unroll=2)

    def out_copy(k):
        return pltpu.make_async_copy(
            ob, out_hbm.at[pl.ds(base_g * CW + k * OBW, OBW)], so)

    fire(0, setA)
    out_copy(0).start()

    def step(k, carry):
        c0 = 2 * k
        fire(c0 + 1, setB)
        wait_gathers(setA)
        out_copy(k).wait()
        combine(setA, 0)
        nxt = jnp.minimum(c0 + 2, NCHUNK - 2)
        fire(nxt, setA)
        wait_gathers(setB)
        combine(setB, 1)
        out_copy(k).start()
        return carry

    lax.fori_loop(0, PAIRS, step, 0)

    out_copy(0).wait()
    wait_gathers(setA)


def _scratch_set():
    return (
        [pltpu.VMEM((CHUNK,), jnp.int32) for _ in range(2)]
        + [pltpu.VMEM((CHUNK,), jnp.float32) for _ in range(4)]
        + [pltpu.VMEM((CHUNK, 2 * CW), jnp.int32) for _ in range(2)]
    )


_sc_sample = pl.kernel(
    _sc_body,
    out_type=jax.ShapeDtypeStruct((TOTAL * CW,), jnp.int32),
    mesh=plsc.VectorSubcoreMesh(
        core_axis_name="c", subcore_axis_name="s",
        num_cores=NC, num_subcores=NS),
    compiler_params=pltpu.CompilerParams(needs_layout_passes=False),
    scratch_types=(
        [pltpu.VMEM((PER_TILE,), jnp.float32),
         pltpu.VMEM((PER_TILE,), jnp.float32)]
        + _scratch_set()
        + _scratch_set()
        + [pltpu.VMEM((OBW,), jnp.int32)]
        + [pltpu.SemaphoreType.DMA] * 5
    ),
)


def _to_bf16_bits(x):
    u = jax.lax.bitcast_convert_type(x, jnp.int32)
    lsb = jax.lax.shift_right_logical(u, 16) & 1
    return jax.lax.shift_right_logical(u + 0x7FFF + lsb, 16)


@jax.jit
def kernel(img, points):
    n, c, h, w = img.shape
    tl = jnp.pad(
        img.transpose(0, 2, 3, 1), ((0, 0), (1, 1), (1, 1), (0, 0)))
    words = _to_bf16_bits(tl[..., :CW]) | jnp.left_shift(
        _to_bf16_bits(tl[..., CW:]), 16)
    fw = words.reshape(-1)
    even = fw.reshape(NPAIR, 2 * CW)
    odd = jnp.concatenate(
        [fw[CW:], jnp.zeros((CW,), jnp.int32)]).reshape(NPAIR, 2 * CW)
    table = jnp.concatenate([even, odd], axis=0)
    xs = points[..., 0].reshape(-1)
    ys = points[..., 1].reshape(-1)
    out_w = _sc_sample(table, xs, ys).reshape(n, h, w, CW)
    lo = jax.lax.bitcast_convert_type(
        jnp.left_shift(out_w, 16), jnp.float32)
    hi = jax.lax.bitcast_convert_type(
        out_w & jnp.int32(-65536), jnp.float32)
    out_t = jnp.concatenate([lo, hi], axis=-1)
    return out_t.transpose(0, 3, 1, 2)
